# baseline (speedup 1.0000x reference)
import jax
import jax.numpy as jnp
from jax.experimental import pallas as pl
from jax.experimental.pallas import tpu as pltpu

B, S, D = 4, 2048, 1024
NUM_STATES = 64
S_BLK = 512


def _add_kernel(idx_ref, h_ref, cv_ref, o_ref):
    o_ref[...] = h_ref[...] + cv_ref[...]


def kernel(hidden_states, affective_state_indices, control_vectors):
    idx = affective_state_indices.astype(jnp.int32)
    cv3 = control_vectors.reshape(NUM_STATES, 1, D)
    grid = (B, S // S_BLK)
    return pl.pallas_call(
        _add_kernel,
        grid_spec=pltpu.PrefetchScalarGridSpec(
            num_scalar_prefetch=1,
            grid=grid,
            in_specs=[
                pl.BlockSpec((1, S_BLK, D), lambda b, s, idx_ref: (b, s, 0)),
                pl.BlockSpec((1, 1, D), lambda b, s, idx_ref: (idx_ref[b], 0, 0)),
            ],
            out_specs=pl.BlockSpec((1, S_BLK, D), lambda b, s, idx_ref: (b, s, 0)),
        ),
        out_shape=jax.ShapeDtypeStruct((B, S, D), jnp.float32),
    )(idx, hidden_states, cv3)
